# baseline (speedup 1.0000x reference)
import functools

import jax
import jax.numpy as jnp
from jax.experimental import pallas as pl
from jax.experimental.pallas import tpu as pltpu


def _fused_kernel(src_ref, dst_ref, ctx_ref, bin_ref, q_ref, V_ref, U_ref,
                  b_ref, W1_ref, W2_ref, out_ref, adj_ref, adjt_ref, maxp_ref,
                  ctxb_ref, bint_ref, *, nb, chunks_per_step):
    k = pl.program_id(0)
    nsteps = pl.num_programs(0)
    m_tok, d2 = ctx_ref.shape
    n = adj_ref.shape[0]

    dn_c0 = (((0,), (0,)), ((), ()))
    dn_c1 = (((1,), (1,)), ((), ()))
    dn_mm = (((1,), (0,)), ((), ()))

    @pl.when(k == 0)
    def _init():
        adj_ref[...] = jnp.zeros_like(adj_ref)
        adjt_ref[...] = jnp.zeros_like(adjt_ref)
        ctxb_ref[...] = ctx_ref[...].astype(jnp.bfloat16)
        bint_ref[...] = bin_ref[...].astype(jnp.bfloat16).T

    ctx = ctxb_ref[...]

    src = src_ref[...]
    dst = dst_ref[...]
    iota = jax.lax.broadcasted_iota(jnp.int32, (src.shape[0], n), 1)
    oh_s = (src == iota).astype(jnp.bfloat16)
    oh_d = (dst == iota).astype(jnp.bfloat16)
    adj_ref[...] += jax.lax.dot_general(
        oh_s, oh_d, dn_c0, preferred_element_type=jnp.float32)
    adjt_ref[...] += jax.lax.dot_general(
        oh_d, oh_s, dn_c0, preferred_element_type=jnp.float32)

    def body(i, carry):
        g = bint_ref[pl.ds(i * nb, nb), :]
        mx = jnp.max(g[:, :, None] * ctx[None, :, :], axis=1)
        maxp_ref[pl.ds(i * nb, nb), :] = mx
        return carry

    jax.lax.fori_loop(k * chunks_per_step, (k + 1) * chunks_per_step, body, 0)

    @pl.when(k == nsteps - 1)
    def _finish():
        adj = jnp.minimum(adj_ref[...], 1.0)
        adjt = jnp.minimum(adjt_ref[...], 1.0)

        mean_p = jax.lax.dot_general(
            bint_ref[...], ctx, dn_mm,
            preferred_element_type=jnp.float32) * (1.0 / m_tok)
        max_p = maxp_ref[...].astype(jnp.float32)
        ent = jnp.concatenate([mean_p, max_p], axis=1)

        q = jnp.mean(q_ref[...], axis=0, keepdims=True)
        qV = jax.lax.dot_general(
            q, V_ref[...], dn_mm, preferred_element_type=jnp.float32)
        gammas = jax.lax.dot_general(
            ent, qV, dn_c1, preferred_element_type=jnp.float32)
        gate = jax.nn.sigmoid(gammas * (1.0 / (d2 ** 0.5)))
        entU = jax.lax.dot_general(
            ent, U_ref[...], dn_c1, preferred_element_type=jnp.float32)
        hidden = gate * entU + b_ref[...]

        s_i_row = jax.lax.dot_general(
            W1_ref[...], hidden, dn_c1,
            preferred_element_type=jnp.float32)
        s_j_col = jax.lax.dot_general(
            hidden, W2_ref[...], dn_c1,
            preferred_element_type=jnp.float32)

        G = s_j_col + s_i_row
        lrelu = jnp.where(G >= 0, G, 0.01 * G)
        F = jnp.where(adjt > 0, jnp.exp(lrelu), 1.0)
        colsum = jnp.sum(F, axis=0, keepdims=True)
        mtx = adj * F * (1.0 / colsum)
        recv = jax.lax.dot_general(
            mtx, hidden, dn_mm, preferred_element_type=jnp.float32)
        out_ref[...] = jnp.maximum(recv, 0.0).T


def kernel(context_emb, query_emb, bin_M, edge_index, V, U, b, W):
    B, M, d2 = context_emb.shape
    N = bin_M.shape[1]
    nE = edge_index.shape[1]
    ctx = context_emb[0]
    src = edge_index[0].reshape(nE, 1)
    dst = edge_index[1].reshape(nE, 1)
    b2 = b.reshape(1, d2)
    W1 = W[:d2, 0].reshape(1, d2)
    W2 = W[d2:, 0].reshape(1, d2)

    EC = 8192
    steps = nE // EC
    nb = 32
    chunks_per_step = (N // nb) // steps

    full = lambda s: pl.BlockSpec(s, lambda k: tuple(0 for _ in s))
    out = pl.pallas_call(
        functools.partial(_fused_kernel, nb=nb,
                          chunks_per_step=chunks_per_step),
        grid=(steps,),
        in_specs=[
            pl.BlockSpec((EC, 1), lambda k: (k, 0)),
            pl.BlockSpec((EC, 1), lambda k: (k, 0)),
            full((M, d2)),
            full((M, N)),
            full((query_emb.shape[0], d2)),
            full((d2, 2 * d2)),
            full((d2, 2 * d2)),
            full((1, d2)),
            full((1, d2)),
            full((1, d2)),
        ],
        out_specs=full((d2, N)),
        out_shape=jax.ShapeDtypeStruct((d2, N), jnp.float32),
        scratch_shapes=[
            pltpu.VMEM((N, N), jnp.float32),
            pltpu.VMEM((N, N), jnp.float32),
            pltpu.VMEM((N, d2), jnp.bfloat16),
            pltpu.VMEM((M, d2), jnp.bfloat16),
            pltpu.VMEM((N, M), jnp.bfloat16),
        ],
        compiler_params=pltpu.CompilerParams(
            dimension_semantics=("arbitrary",)),
    )(src, dst, ctx, bin_M, query_emb, V, U, b2, W1, W2)
    return out


# speedup vs baseline: 1.0183x; 1.0183x over previous
import functools

import jax
import jax.numpy as jnp
from jax.experimental import pallas as pl
from jax.experimental.pallas import tpu as pltpu


def _fused_kernel(src_ref, dst_ref, ctx_ref, bin_ref, q_ref, V_ref, U_ref,
                  b_ref, W1_ref, W2_ref, out_ref, adj_ref, adjt_ref, maxp_ref,
                  ctxb_ref, bint_ref, *, nb, chunks_per_step):
    k = pl.program_id(0)
    nsteps = pl.num_programs(0)
    m_tok, d2 = ctx_ref.shape
    n = adj_ref.shape[0]

    dn_c0 = (((0,), (0,)), ((), ()))
    dn_c1 = (((1,), (1,)), ((), ()))
    dn_mm = (((1,), (0,)), ((), ()))

    @pl.when(k == 0)
    def _init():
        adj_ref[...] = jnp.zeros_like(adj_ref)
        adjt_ref[...] = jnp.zeros_like(adjt_ref)
        ctxb_ref[...] = ctx_ref[...].astype(jnp.bfloat16)
        bint_ref[...] = bin_ref[...].astype(jnp.bfloat16).T

    ctx = ctxb_ref[...]

    src = src_ref[...]
    dst = dst_ref[...]
    iota = jax.lax.broadcasted_iota(jnp.int32, (src.shape[0], n), 1)
    oh_s = (src == iota).astype(jnp.bfloat16)
    oh_d = (dst == iota).astype(jnp.bfloat16)
    adj_ref[...] += jax.lax.dot_general(
        oh_s, oh_d, dn_c0, preferred_element_type=jnp.float32)
    adjt_ref[...] += jax.lax.dot_general(
        oh_d, oh_s, dn_c0, preferred_element_type=jnp.float32)

    def body(i, carry):
        g = bint_ref[pl.ds(i * nb, nb), :]
        mx = jnp.max(g[:, :, None] * ctx[None, :, :], axis=1)
        maxp_ref[pl.ds(i * nb, nb), :] = mx
        return carry

    jax.lax.fori_loop(k * chunks_per_step, (k + 1) * chunks_per_step, body, 0)

    @pl.when(k == nsteps - 1)
    def _finish():
        adj = jnp.minimum(adj_ref[...], 1.0)
        adjt = jnp.minimum(adjt_ref[...], 1.0)

        mean_p = jax.lax.dot_general(
            bint_ref[...], ctx, dn_mm,
            preferred_element_type=jnp.float32) * (1.0 / m_tok)
        max_p = maxp_ref[...].astype(jnp.float32)
        ent = jnp.concatenate([mean_p, max_p], axis=1)

        q = jnp.mean(q_ref[...], axis=0, keepdims=True)
        qV = jax.lax.dot_general(
            q, V_ref[...], dn_mm, preferred_element_type=jnp.float32)
        gammas = jax.lax.dot_general(
            ent, qV, dn_c1, preferred_element_type=jnp.float32)
        gate = jax.nn.sigmoid(gammas * (1.0 / (d2 ** 0.5)))
        entU = jax.lax.dot_general(
            ent, U_ref[...], dn_c1, preferred_element_type=jnp.float32)
        hidden = gate * entU + b_ref[...]

        s_i_row = jax.lax.dot_general(
            W1_ref[...], hidden, dn_c1,
            preferred_element_type=jnp.float32)
        s_j_col = jax.lax.dot_general(
            hidden, W2_ref[...], dn_c1,
            preferred_element_type=jnp.float32)

        G = s_j_col + s_i_row
        lrelu = jnp.where(G >= 0, G, 0.01 * G)
        F = jnp.where(adjt > 0, jnp.exp(lrelu), 1.0)
        colsum = jnp.sum(F, axis=0, keepdims=True)
        mtx = adj * F * (1.0 / colsum)
        recv = jax.lax.dot_general(
            mtx, hidden, dn_mm, preferred_element_type=jnp.float32)
        out_ref[...] = jnp.maximum(recv, 0.0).T


def kernel(context_emb, query_emb, bin_M, edge_index, V, U, b, W):
    B, M, d2 = context_emb.shape
    N = bin_M.shape[1]
    nE = edge_index.shape[1]
    ctx = context_emb[0]
    src = edge_index[0].reshape(nE, 1)
    dst = edge_index[1].reshape(nE, 1)
    b2 = b.reshape(1, d2)
    W1 = W[:d2, 0].reshape(1, d2)
    W2 = W[d2:, 0].reshape(1, d2)

    EC = 4096
    steps = nE // EC
    nb = 64
    chunks_per_step = (N // nb) // steps

    full = lambda s: pl.BlockSpec(s, lambda k: tuple(0 for _ in s))
    out = pl.pallas_call(
        functools.partial(_fused_kernel, nb=nb,
                          chunks_per_step=chunks_per_step),
        grid=(steps,),
        in_specs=[
            pl.BlockSpec((EC, 1), lambda k: (k, 0)),
            pl.BlockSpec((EC, 1), lambda k: (k, 0)),
            full((M, d2)),
            full((M, N)),
            full((query_emb.shape[0], d2)),
            full((d2, 2 * d2)),
            full((d2, 2 * d2)),
            full((1, d2)),
            full((1, d2)),
            full((1, d2)),
        ],
        out_specs=full((d2, N)),
        out_shape=jax.ShapeDtypeStruct((d2, N), jnp.float32),
        scratch_shapes=[
            pltpu.VMEM((N, N), jnp.float32),
            pltpu.VMEM((N, N), jnp.float32),
            pltpu.VMEM((N, d2), jnp.bfloat16),
            pltpu.VMEM((M, d2), jnp.bfloat16),
            pltpu.VMEM((N, M), jnp.bfloat16),
        ],
        compiler_params=pltpu.CompilerParams(
            dimension_semantics=("arbitrary",)),
    )(src, dst, ctx, bin_M, query_emb, V, U, b2, W1, W2)
    return out
